# baseline (speedup 1.0000x reference)
import jax
import jax.numpy as jnp
from jax import lax
from jax.experimental import pallas as pl
from jax.experimental.pallas import tpu as pltpu

_D = 2048
_E = 16
_TOPK = 2
_T = 16384
_BT = 1024


def _router_body(h_ref, wt_ref, b_ref, logits_ref, wts_ref, sel_ref, mask_ref):
    h = h_ref[...]
    wt = wt_ref[...]
    logits = jnp.dot(h, wt, preferred_element_type=jnp.float32) + b_ref[...]
    logits_ref[...] = logits

    e_iota = lax.broadcasted_iota(jnp.int32, (_BT, _E), 1)
    v1 = jnp.max(logits, axis=1, keepdims=True)
    i1 = jnp.min(jnp.where(logits == v1, e_iota, _E), axis=1, keepdims=True)
    neg = jnp.float32(-jnp.inf)
    l2 = jnp.where(e_iota == i1, neg, logits)
    v2 = jnp.max(l2, axis=1, keepdims=True)
    i2 = jnp.min(jnp.where(l2 == v2, e_iota, _E), axis=1, keepdims=True)

    e2 = jnp.exp(v2 - v1)
    denom = 1.0 + e2
    w1 = 1.0 / denom
    w2 = e2 / denom
    wts_ref[...] = jnp.concatenate([w1, w2], axis=1)
    sel_ref[...] = jnp.concatenate([i1, i2], axis=1)

    r_iota = lax.broadcasted_iota(jnp.int32, (_BT, _E * _TOPK), 1)
    sel_r = jnp.where((r_iota & 1) == 0, i1, i2)
    mask_ref[...] = (sel_r == (r_iota >> 1)).astype(jnp.int32)


def kernel(hidden_states, W, b):
    wt = W.T
    b2 = b.reshape(1, _E)
    grid = (_T // _BT,)
    logits, wts, sel, mask_t = pl.pallas_call(
        _router_body,
        grid=grid,
        in_specs=[
            pl.BlockSpec((_BT, _D), lambda i: (i, 0)),
            pl.BlockSpec((_D, _E), lambda i: (0, 0)),
            pl.BlockSpec((1, _E), lambda i: (0, 0)),
        ],
        out_specs=[
            pl.BlockSpec((_BT, _E), lambda i: (i, 0)),
            pl.BlockSpec((_BT, _TOPK), lambda i: (i, 0)),
            pl.BlockSpec((_BT, _TOPK), lambda i: (i, 0)),
            pl.BlockSpec((_BT, _E * _TOPK), lambda i: (i, 0)),
        ],
        out_shape=[
            jax.ShapeDtypeStruct((_T, _E), jnp.float32),
            jax.ShapeDtypeStruct((_T, _TOPK), jnp.float32),
            jax.ShapeDtypeStruct((_T, _TOPK), jnp.int32),
            jax.ShapeDtypeStruct((_T, _E * _TOPK), jnp.int32),
        ],
        compiler_params=pltpu.CompilerParams(
            dimension_semantics=("arbitrary",),
        ),
    )(hidden_states, wt, b2)
    expert_mask = mask_t.T.reshape(_E, _TOPK, _T)
    return (logits, wts, sel, expert_mask)


# speedup vs baseline: 1.6527x; 1.6527x over previous
import jax
import jax.numpy as jnp
from jax import lax
from jax.experimental import pallas as pl
from jax.experimental.pallas import tpu as pltpu

_D = 2048
_E = 16
_TOPK = 2
_T = 16384
_BT = 1024


def _router_body(h_ref, wt_ref, b_ref, logits_ref, wts_ref, sel_ref, mask_ref):
    h = h_ref[...]
    wt = wt_ref[...]
    logits = jnp.dot(h, wt, preferred_element_type=jnp.float32)
    lt = logits.T + b_ref[...]
    logits_ref[...] = lt

    e_iota = lax.broadcasted_iota(jnp.int32, (_E, _BT), 0)
    v1 = jnp.max(lt, axis=0, keepdims=True)
    i1 = jnp.min(jnp.where(lt == v1, e_iota, _E), axis=0, keepdims=True)
    l2 = jnp.where(e_iota == i1, jnp.float32(-jnp.inf), lt)
    v2 = jnp.max(l2, axis=0, keepdims=True)
    i2 = jnp.min(jnp.where(l2 == v2, e_iota, _E), axis=0, keepdims=True)

    e2 = jnp.exp(v2 - v1)
    denom = 1.0 + e2
    wts_ref[...] = jnp.concatenate([1.0 / denom, e2 / denom], axis=0)
    sel_ref[...] = jnp.concatenate([i1, i2], axis=0)

    r_iota = lax.broadcasted_iota(jnp.int32, (_E * _TOPK, _BT), 0)
    sel_r = jnp.where((r_iota & 1) == 0, i1, i2)
    mask_ref[...] = (sel_r == (r_iota >> 1)).astype(jnp.int32)


def kernel(hidden_states, W, b):
    wt = W.T
    b2 = b.reshape(_E, 1)
    grid = (_T // _BT,)
    logits_t, wts_t, sel_t, mask_t = pl.pallas_call(
        _router_body,
        grid=grid,
        in_specs=[
            pl.BlockSpec((_BT, _D), lambda i: (i, 0)),
            pl.BlockSpec((_D, _E), lambda i: (0, 0)),
            pl.BlockSpec((_E, 1), lambda i: (0, 0)),
        ],
        out_specs=[
            pl.BlockSpec((_E, _BT), lambda i: (0, i)),
            pl.BlockSpec((_TOPK, _BT), lambda i: (0, i)),
            pl.BlockSpec((_TOPK, _BT), lambda i: (0, i)),
            pl.BlockSpec((_E * _TOPK, _BT), lambda i: (0, i)),
        ],
        out_shape=[
            jax.ShapeDtypeStruct((_E, _T), jnp.float32),
            jax.ShapeDtypeStruct((_TOPK, _T), jnp.float32),
            jax.ShapeDtypeStruct((_TOPK, _T), jnp.int32),
            jax.ShapeDtypeStruct((_E * _TOPK, _T), jnp.int32),
        ],
        compiler_params=pltpu.CompilerParams(
            dimension_semantics=("arbitrary",),
        ),
    )(hidden_states, wt, b2)
    return (logits_t.T, wts_t.T, sel_t.T, mask_t.reshape(_E, _TOPK, _T))
